# baseline (speedup 1.0000x reference)
import jax
import jax.numpy as jnp
from jax.experimental import pallas as pl
from jax.experimental.pallas import tpu as pltpu

_K = 512
_STEPS = 16
_LMAX = 40
_CAP = _STEPS + 2


def _peaks_body(x_ref, score_ref, bcls_ref, argc_ref):
    x = x_ref[0]
    C, H, W = x.shape
    m = x[0]
    for c in range(1, C):
        m = jnp.maximum(m, x[c])
    es = []
    s = None
    for c in range(C):
        e = jnp.exp(x[c] - m)
        es.append(e)
        s = e if s is None else s + e
    inv = 1.0 / s
    ninf = jnp.float32(-jnp.inf)
    best = bcls = abest = agc = None
    for c in range(C):
        p = es[c] * inv
        lf = jnp.concatenate([p[:, 1:], jnp.full((H, 1), ninf, p.dtype)], axis=1)
        rt = jnp.concatenate([jnp.full((H, 1), ninf, p.dtype), p[:, :-1]], axis=1)
        mw = jnp.maximum(p, jnp.maximum(lf, rt))
        up = jnp.concatenate([mw[1:], jnp.full((1, W), ninf, p.dtype)], axis=0)
        dn = jnp.concatenate([jnp.full((1, W), ninf, p.dtype), mw[:-1]], axis=0)
        lm = jnp.maximum(mw, jnp.maximum(up, dn))
        pk = (p == lm) & (p > 0.5)
        msk = jnp.where(pk, p, 0.0)
        xc = x[c]
        if best is None:
            best = msk
            bcls = jnp.zeros(msk.shape, jnp.int32)
            abest = xc
            agc = jnp.zeros(msk.shape, jnp.int32)
        else:
            u = msk > best
            bcls = jnp.where(u, c, bcls)
            best = jnp.where(u, msk, best)
            au = xc > abest
            agc = jnp.where(au, c, agc)
            abest = jnp.where(au, xc, abest)
    score_ref[0] = best
    bcls_ref[0] = bcls
    argc_ref[0] = agc


def _dense(xt):
    B, C, H, W = xt.shape
    return pl.pallas_call(
        _peaks_body,
        grid=(B,),
        in_specs=[pl.BlockSpec((1, C, H, W), lambda b: (b, 0, 0, 0))],
        out_specs=[
            pl.BlockSpec((1, H, W), lambda b: (b, 0, 0)),
            pl.BlockSpec((1, H, W), lambda b: (b, 0, 0)),
            pl.BlockSpec((1, H, W), lambda b: (b, 0, 0)),
        ],
        out_shape=[
            jax.ShapeDtypeStruct((B, H, W), jnp.float32),
            jax.ShapeDtypeStruct((B, H, W), jnp.int32),
            jax.ShapeDtypeStruct((B, H, W), jnp.int32),
        ],
    )(xt)


def _side_probe(gp, cid, argcls_b, sp_map, sl_map, H, W):
    gi = jnp.floor(gp).astype(jnp.int32)
    in_g = (gi[0] >= 0) & (gi[0] < H) & (gi[1] >= 0) & (gi[1] < W)
    g0 = jnp.clip(gi[0], 0, H - 1)
    g1 = jnp.clip(gi[1], 0, W - 1)
    sp = gp + 0.5 + sp_map[g0, g1]
    si = jnp.floor(sp).astype(jnp.int32)
    in_s = (si[0] >= 0) & (si[0] < H) & (si[1] >= 0) & (si[1] < W)
    s0 = jnp.clip(si[0], 0, H - 1)
    s1 = jnp.clip(si[1], 0, W - 1)
    ok = in_g & in_s & (argcls_b[s0, s1] == cid) & (sl_map[s0, s1] <= 0.0)
    return sp, ok


def _grow(pts, n, alive, cid, argcls_b, sp_maps, sl_maps, H, W):
    last = pts[jnp.clip(n - 1, 0, _CAP - 1)]
    prev = pts[jnp.clip(n - 2, 0, _CAP - 1)]
    direction = last - prev
    lp, lok = _side_probe(last, cid, argcls_b, sp_maps[0], sl_maps[0], H, W)
    rp, rok = _side_probe(last, cid, argcls_b, sp_maps[1], sl_maps[1], H, W)
    ld = jnp.where(lok, lp[0] * direction[0] + lp[1] * direction[1], -1.0)
    rd = jnp.where(rok, rp[0] * direction[0] + rp[1] * direction[1], -1.0)
    take_l = lok & (ld > 0) & (ld > rd)
    take_r = rok & (rd > 0) & (rd > ld)
    appended = take_l | take_r
    do_append = alive & appended
    new_pt = jnp.where(take_l, lp, rp)
    slot = jnp.clip(n, 0, _CAP - 1)
    pts = pts.at[slot].set(jnp.where(do_append, new_pt, pts[slot]))
    n = n + jnp.where(do_append, 1, 0).astype(jnp.int32)
    alive = alive & appended
    return pts, n, alive


def _trace(ys, xs, cls, valid, argcls, cp, sp0, sp1, sl0, sl1, H, W):
    def trace_peak(py, px, cid, keep, argcls_b, cp_b, sp0_b, sp1_b, sl0_b, sl1_b):
        gp = jnp.stack([py, px]).astype(jnp.float32)
        start = gp + cp_b[py, px]
        lpt, lok = _side_probe(gp, cid, argcls_b, sp0_b, sl0_b, H, W)
        rpt, rok = _side_probe(gp, cid, argcls_b, sp1_b, sl1_b, H, W)

        def mk(p0, ok0):
            pts = jnp.zeros((_CAP, 2), jnp.float32)
            pts = pts.at[0].set(start)
            pts = pts.at[1].set(jnp.where(ok0, p0, 0.0))
            n = jnp.where(ok0, 2, 1).astype(jnp.int32)
            return pts, n, ok0

        def body(_, carry):
            (a_pts, a_n, a_al), (b_pts, b_n, b_al) = carry
            left = _grow(a_pts, a_n, a_al, cid, argcls_b, (sp0_b, sp1_b), (sl0_b, sl1_b), H, W)
            right = _grow(b_pts, b_n, b_al, cid, argcls_b, (sp0_b, sp1_b), (sl0_b, sl1_b), H, W)
            return left, right

        (pl_, nl, _), (pr_, nr, _) = jax.lax.fori_loop(
            0, _STEPS, body, (mk(lpt, lok), mk(rpt, rok))
        )
        total = nl + nr - 1
        j = jnp.arange(_LMAX)
        from_l = j < nl
        pt = jnp.where(
            from_l[:, None],
            pl_[jnp.clip(nl - 1 - j, 0, _CAP - 1)],
            pr_[jnp.clip(j - nl + 1, 0, _CAP - 1)],
        )
        line = jnp.where((j < total)[:, None] & keep, pt, 0.0)
        length = jnp.where(keep, total, 0).astype(jnp.int32)
        return line, length

    per_peak = jax.vmap(trace_peak, in_axes=(0, 0, 0, 0, None, None, None, None, None, None))
    per_batch = jax.vmap(per_peak)
    return per_batch(ys, xs, cls, valid, argcls, cp, sp0, sp1, sl0, sl1)


def kernel(segm_logit, side_logit0, side_logit1, center_point, side_points0, side_points1):
    B, H, W, C = segm_logit.shape
    xt = jnp.transpose(segm_logit, (0, 3, 1, 2))
    best_score, best_class, argcls = _dense(xt)

    HW = H * W
    S = 16
    sub_s, sub_i = jax.lax.top_k(best_score.reshape(B * S, HW // S), _K)
    base = (jnp.arange(S, dtype=jnp.int32) * (HW // S)).repeat(_K)
    cand_i = (sub_i.reshape(B, S * _K) + base[None, :]).astype(jnp.int32)
    top_s, tj = jax.lax.top_k(sub_s.reshape(B, S * _K), _K)
    top_i = jnp.take_along_axis(cand_i, tj, axis=1)
    ys = top_i // W
    xs = top_i % W
    cls = jnp.take_along_axis(best_class.reshape(B, H * W), top_i, axis=1)
    valid = top_s > 0.0

    sl0 = side_logit0[..., 0]
    sl1 = side_logit1[..., 0]
    lines, lens = _trace(
        ys, xs, cls, valid, argcls, center_point, side_points0, side_points1, sl0, sl1, H, W
    )
    points = jnp.stack([ys, xs], axis=-1).astype(jnp.int32)
    return (points, cls.astype(jnp.int32), top_s, lines, lens)


# speedup vs baseline: 10.7180x; 10.7180x over previous
import functools

import jax
import jax.numpy as jnp
from jax import lax
from jax.experimental import pallas as pl
from jax.experimental.pallas import tpu as pltpu
from jax.experimental.pallas import tpu_sc as plsc

_K = 512
_STEPS = 16
_LMAX = 40
_CAP = _STEPS + 2


def _peaks_body(x_ref, score_ref, bcls_ref, argc_ref):
    x = x_ref[0]
    C, H, W = x.shape
    m = x[0]
    for c in range(1, C):
        m = jnp.maximum(m, x[c])
    es = []
    s = None
    for c in range(C):
        e = jnp.exp(x[c] - m)
        es.append(e)
        s = e if s is None else s + e
    inv = 1.0 / s
    ninf = jnp.float32(-jnp.inf)
    best = bcls = abest = agc = None
    for c in range(C):
        p = es[c] * inv
        lf = jnp.concatenate([p[:, 1:], jnp.full((H, 1), ninf, p.dtype)], axis=1)
        rt = jnp.concatenate([jnp.full((H, 1), ninf, p.dtype), p[:, :-1]], axis=1)
        mw = jnp.maximum(p, jnp.maximum(lf, rt))
        up = jnp.concatenate([mw[1:], jnp.full((1, W), ninf, p.dtype)], axis=0)
        dn = jnp.concatenate([jnp.full((1, W), ninf, p.dtype), mw[:-1]], axis=0)
        lm = jnp.maximum(mw, jnp.maximum(up, dn))
        pk = (p == lm) & (p > 0.5)
        msk = jnp.where(pk, p, 0.0)
        xc = x[c]
        if best is None:
            best = msk
            bcls = jnp.zeros(msk.shape, jnp.int32)
            abest = xc
            agc = jnp.zeros(msk.shape, jnp.int32)
        else:
            u = msk > best
            bcls = jnp.where(u, c, bcls)
            best = jnp.where(u, msk, best)
            au = xc > abest
            agc = jnp.where(au, c, agc)
            abest = jnp.where(au, xc, abest)
    score_ref[0] = best
    bcls_ref[0] = bcls
    argc_ref[0] = agc


def _dense(xt):
    B, C, H, W = xt.shape
    return pl.pallas_call(
        _peaks_body,
        grid=(B,),
        in_specs=[pl.BlockSpec((1, C, H, W), lambda b: (b, 0, 0, 0))],
        out_specs=[
            pl.BlockSpec((1, H, W), lambda b: (b, 0, 0)),
            pl.BlockSpec((1, H, W), lambda b: (b, 0, 0)),
            pl.BlockSpec((1, H, W), lambda b: (b, 0, 0)),
        ],
        out_shape=[
            jax.ShapeDtypeStruct((B, H, W), jnp.float32),
            jax.ShapeDtypeStruct((B, H, W), jnp.int32),
            jax.ShapeDtypeStruct((B, H, W), jnp.int32),
        ],
    )(xt)


def _side_probe(gp, cid, argcls_b, sp_map, sl_map, H, W):
    gi = jnp.floor(gp).astype(jnp.int32)
    in_g = (gi[0] >= 0) & (gi[0] < H) & (gi[1] >= 0) & (gi[1] < W)
    g0 = jnp.clip(gi[0], 0, H - 1)
    g1 = jnp.clip(gi[1], 0, W - 1)
    sp = gp + 0.5 + sp_map[g0, g1]
    si = jnp.floor(sp).astype(jnp.int32)
    in_s = (si[0] >= 0) & (si[0] < H) & (si[1] >= 0) & (si[1] < W)
    s0 = jnp.clip(si[0], 0, H - 1)
    s1 = jnp.clip(si[1], 0, W - 1)
    ok = in_g & in_s & (argcls_b[s0, s1] == cid) & (sl_map[s0, s1] <= 0.0)
    return sp, ok


def _grow(pts, n, alive, cid, argcls_b, sp_maps, sl_maps, H, W):
    last = pts[jnp.clip(n - 1, 0, _CAP - 1)]
    prev = pts[jnp.clip(n - 2, 0, _CAP - 1)]
    direction = last - prev
    lp, lok = _side_probe(last, cid, argcls_b, sp_maps[0], sl_maps[0], H, W)
    rp, rok = _side_probe(last, cid, argcls_b, sp_maps[1], sl_maps[1], H, W)
    ld = jnp.where(lok, lp[0] * direction[0] + lp[1] * direction[1], -1.0)
    rd = jnp.where(rok, rp[0] * direction[0] + rp[1] * direction[1], -1.0)
    take_l = lok & (ld > 0) & (ld > rd)
    take_r = rok & (rd > 0) & (rd > ld)
    appended = take_l | take_r
    do_append = alive & appended
    new_pt = jnp.where(take_l, lp, rp)
    slot = jnp.clip(n, 0, _CAP - 1)
    pts = pts.at[slot].set(jnp.where(do_append, new_pt, pts[slot]))
    n = n + jnp.where(do_append, 1, 0).astype(jnp.int32)
    alive = alive & appended
    return pts, n, alive


def _trace(ys, xs, cls, valid, argcls, cp, sp0, sp1, sl0, sl1, H, W):
    def trace_peak(py, px, cid, keep, argcls_b, cp_b, sp0_b, sp1_b, sl0_b, sl1_b):
        gp = jnp.stack([py, px]).astype(jnp.float32)
        start = gp + cp_b[py, px]
        lpt, lok = _side_probe(gp, cid, argcls_b, sp0_b, sl0_b, H, W)
        rpt, rok = _side_probe(gp, cid, argcls_b, sp1_b, sl1_b, H, W)

        def mk(p0, ok0):
            pts = jnp.zeros((_CAP, 2), jnp.float32)
            pts = pts.at[0].set(start)
            pts = pts.at[1].set(jnp.where(ok0, p0, 0.0))
            n = jnp.where(ok0, 2, 1).astype(jnp.int32)
            return pts, n, ok0

        def body(_, carry):
            (a_pts, a_n, a_al), (b_pts, b_n, b_al) = carry
            left = _grow(a_pts, a_n, a_al, cid, argcls_b, (sp0_b, sp1_b), (sl0_b, sl1_b), H, W)
            right = _grow(b_pts, b_n, b_al, cid, argcls_b, (sp0_b, sp1_b), (sl0_b, sl1_b), H, W)
            return left, right

        (pl_, nl, _), (pr_, nr, _) = jax.lax.fori_loop(
            0, _STEPS, body, (mk(lpt, lok), mk(rpt, rok))
        )
        total = nl + nr - 1
        j = jnp.arange(_LMAX)
        from_l = j < nl
        pt = jnp.where(
            from_l[:, None],
            pl_[jnp.clip(nl - 1 - j, 0, _CAP - 1)],
            pr_[jnp.clip(j - nl + 1, 0, _CAP - 1)],
        )
        line = jnp.where((j < total)[:, None] & keep, pt, 0.0)
        length = jnp.where(keep, total, 0).astype(jnp.int32)
        return line, length

    per_peak = jax.vmap(trace_peak, in_axes=(0, 0, 0, 0, None, None, None, None, None, None))
    per_batch = jax.vmap(per_peak)
    return per_batch(ys, xs, cls, valid, argcls, cp, sp0, sp1, sl0, sl1)


def _sc_trace(H, W, npeak, nbatch):
    HW = H * W
    per_tile = npeak // 32
    ng = per_tile // 16
    cap1 = _CAP - 1

    mesh = plsc.VectorSubcoreMesh(core_axis_name="c", subcore_axis_name="s")
    f32 = jnp.float32
    i32 = jnp.int32

    @functools.partial(
        pl.kernel,
        mesh=mesh,
        compiler_params=pltpu.CompilerParams(needs_layout_passes=False),
        out_type=[
            jax.ShapeDtypeStruct((npeak * 2 * _LMAX,), f32),
            jax.ShapeDtypeStruct((npeak,), i32),
        ],
        scratch_types=(
            [pltpu.VMEM((per_tile,), i32) for _ in range(4)]
            + [pltpu.VMEM((per_tile,), i32) for _ in range(6)]
            + [pltpu.VMEM((per_tile,), f32) for _ in range(8)]
            + [pltpu.VMEM((per_tile,), i32) for _ in range(4)]
            + [pltpu.VMEM((per_tile,), f32) for _ in range(4)]
            + [pltpu.VMEM((_CAP * 2 * per_tile,), f32) for _ in range(2)]
            + [pltpu.VMEM((per_tile * 2 * _LMAX,), f32)]
            + [pltpu.VMEM((per_tile,), i32)]
            + [pltpu.SemaphoreType.DMA]
        ),
    )
    def tracer(ysr, xsr, clsr, vldr, acr, cpyr, cpxr, s0yr, s0xr, s1yr, s1xr,
               sl0r, sl1r, linesr, lensr,
               pky, pkx, pkc, pkv, ixL, ixR, jA, jB, jC, jD,
               ay0, ax0, ay1, ax1, by0, bx0, by1, bx1,
               cA, cB, cC, cD, sA, sB, sC2, sD,
               ptsL, ptsR, lineb, lenv, sem):
        lane = lax.broadcasted_iota(i32, (16,), 0)

        def fl(x):
            t = x.astype(i32)
            return jnp.where(x < t.astype(f32), t - 1, t)

        def grp(ref, g):
            return ref[pl.ds(g * 16, 16)]

        def drain(descs):
            for d in descs:
                d.wait()

        wid = lax.axis_index("c") * 16 + lax.axis_index("s")
        p0 = wid * per_tile
        moff = (wid // (32 // nbatch)) * HW

        pltpu.sync_copy(ysr.at[pl.ds(p0, per_tile)], pky)
        pltpu.sync_copy(xsr.at[pl.ds(p0, per_tile)], pkx)
        pltpu.sync_copy(clsr.at[pl.ds(p0, per_tile)], pkc)
        pltpu.sync_copy(vldr.at[pl.ds(p0, per_tile)], pkv)

        for g in range(ng):
            py = grp(pky, g)
            px = grp(pkx, g)
            ixL[pl.ds(g * 16, 16)] = moff + py * W + px
        drain([
            pltpu.async_copy(cpyr.at[ixL], ay0, sem),
            pltpu.async_copy(cpxr.at[ixL], ax0, sem),
            pltpu.async_copy(s0yr.at[ixL], ay1, sem),
            pltpu.async_copy(s0xr.at[ixL], ax1, sem),
            pltpu.async_copy(s1yr.at[ixL], by0, sem),
            pltpu.async_copy(s1xr.at[ixL], bx0, sem),
        ])
        init = []
        for g in range(ng):
            fy = grp(pky, g).astype(f32)
            fx = grp(pkx, g).astype(f32)
            sty = fy + grp(ay0, g)
            stx = fx + grp(ax0, g)
            lpY = fy + 0.5 + grp(ay1, g)
            lpX = fx + 0.5 + grp(ax1, g)
            rpY = fy + 0.5 + grp(by0, g)
            rpX = fx + 0.5 + grp(bx0, g)
            liy = fl(lpY)
            lix = fl(lpX)
            lin = (liy >= 0) & (liy < H) & (lix >= 0) & (lix < W)
            jA[pl.ds(g * 16, 16)] = moff + jnp.clip(liy, 0, H - 1) * W + jnp.clip(lix, 0, W - 1)
            riy = fl(rpY)
            rix = fl(rpX)
            rin = (riy >= 0) & (riy < H) & (rix >= 0) & (rix < W)
            jB[pl.ds(g * 16, 16)] = moff + jnp.clip(riy, 0, H - 1) * W + jnp.clip(rix, 0, W - 1)
            init.append((sty, stx, lpY, lpX, rpY, rpX, lin, rin))
        drain([
            pltpu.async_copy(acr.at[jA], cA, sem),
            pltpu.async_copy(sl0r.at[jA], sA, sem),
            pltpu.async_copy(acr.at[jB], cB, sem),
            pltpu.async_copy(sl1r.at[jB], sB, sem),
        ])
        nL0, aL0, nR0, aR0 = [], [], [], []
        for g in range(ng):
            sty, stx, lpY, lpX, rpY, rpX, lin, rin = init[g]
            cid = grp(pkc, g)
            lok = lin & (grp(cA, g) == cid) & (grp(sA, g) <= 0.0)
            rok = rin & (grp(cB, g) == cid) & (grp(sB, g) <= 0.0)
            ptsL[pl.ds(g * 16, 16)] = sty
            ptsL[pl.ds(per_tile + g * 16, 16)] = stx
            ptsL[pl.ds(2 * per_tile + g * 16, 16)] = jnp.where(lok, lpY, 0.0)
            ptsL[pl.ds(3 * per_tile + g * 16, 16)] = jnp.where(lok, lpX, 0.0)
            ptsR[pl.ds(g * 16, 16)] = sty
            ptsR[pl.ds(per_tile + g * 16, 16)] = stx
            ptsR[pl.ds(2 * per_tile + g * 16, 16)] = jnp.where(rok, rpY, 0.0)
            ptsR[pl.ds(3 * per_tile + g * 16, 16)] = jnp.where(rok, rpX, 0.0)
            nL0.append(jnp.where(lok, 2, 1).astype(i32))
            aL0.append(lok)
            nR0.append(jnp.where(rok, 2, 1).astype(i32))
            aR0.append(rok)

        stride = 2 * per_tile

        def step(_, carry):
            nL, aL, nR, aR = carry
            stash = {}
            for cname, ns, ix, ptsC in (("L", nL, ixL, ptsL), ("R", nR, ixR, ptsR)):
                for g in range(ng):
                    n = ns[g]
                    s1_ = jnp.clip(n - 1, 0, cap1)
                    s2_ = jnp.clip(n - 2, 0, cap1)
                    gl = g * 16 + lane
                    lastY = plsc.load_gather(ptsC, [s1_ * stride + gl])
                    lastX = plsc.load_gather(ptsC, [s1_ * stride + per_tile + gl])
                    prevY = plsc.load_gather(ptsC, [s2_ * stride + gl])
                    prevX = plsc.load_gather(ptsC, [s2_ * stride + per_tile + gl])
                    dY = lastY - prevY
                    dX = lastX - prevX
                    giy = fl(lastY)
                    gix = fl(lastX)
                    ing = (giy >= 0) & (giy < H) & (gix >= 0) & (gix < W)
                    ix[pl.ds(g * 16, 16)] = (
                        moff + jnp.clip(giy, 0, H - 1) * W + jnp.clip(gix, 0, W - 1)
                    )
                    stash[(cname, g)] = (lastY, lastX, dY, dX, ing)
            drain([
                pltpu.async_copy(s0yr.at[ixL], ay0, sem),
                pltpu.async_copy(s0xr.at[ixL], ax0, sem),
                pltpu.async_copy(s1yr.at[ixL], ay1, sem),
                pltpu.async_copy(s1xr.at[ixL], ax1, sem),
                pltpu.async_copy(s0yr.at[ixR], by0, sem),
                pltpu.async_copy(s0xr.at[ixR], bx0, sem),
                pltpu.async_copy(s1yr.at[ixR], by1, sem),
                pltpu.async_copy(s1xr.at[ixR], bx1, sem),
            ])
            stash2 = {}
            for cname, srcs, jref in (
                ("L0", (ay0, ax0), jA),
                ("L1", (ay1, ax1), jB),
                ("R0", (by0, bx0), jC),
                ("R1", (by1, bx1), jD),
            ):
                for g in range(ng):
                    lastY, lastX, dY, dX, ing = stash[(cname[0], g)]
                    spY = lastY + 0.5 + grp(srcs[0], g)
                    spX = lastX + 0.5 + grp(srcs[1], g)
                    siy = fl(spY)
                    six = fl(spX)
                    ins = ing & (siy >= 0) & (siy < H) & (six >= 0) & (six < W)
                    jref[pl.ds(g * 16, 16)] = (
                        moff + jnp.clip(siy, 0, H - 1) * W + jnp.clip(six, 0, W - 1)
                    )
                    stash2[(cname, g)] = (spY, spX, ins)
            drain([
                pltpu.async_copy(acr.at[jA], cA, sem),
                pltpu.async_copy(sl0r.at[jA], sA, sem),
                pltpu.async_copy(acr.at[jB], cB, sem),
                pltpu.async_copy(sl1r.at[jB], sB, sem),
                pltpu.async_copy(acr.at[jC], cC, sem),
                pltpu.async_copy(sl0r.at[jC], sC2, sem),
                pltpu.async_copy(acr.at[jD], cD, sem),
                pltpu.async_copy(sl1r.at[jD], sD, sem),
            ])
            out = {}
            for cname, ns, als, ptsC, refs0, refs1 in (
                ("L", nL, aL, ptsL, (cA, sA), (cB, sB)),
                ("R", nR, aR, ptsR, (cC, sC2), (cD, sD)),
            ):
                nn, na = [], []
                for g in range(ng):
                    cid = grp(pkc, g)
                    lastY, lastX, dY, dX, ing = stash[(cname, g)]
                    spY0, spX0, ins0 = stash2[(cname + "0", g)]
                    spY1, spX1, ins1 = stash2[(cname + "1", g)]
                    lok = ins0 & (grp(refs0[0], g) == cid) & (grp(refs0[1], g) <= 0.0)
                    rok = ins1 & (grp(refs1[0], g) == cid) & (grp(refs1[1], g) <= 0.0)
                    ld = jnp.where(lok, spY0 * dY + spX0 * dX, -1.0)
                    rd = jnp.where(rok, spY1 * dY + spX1 * dX, -1.0)
                    tl = lok & (ld > 0) & (ld > rd)
                    tr = rok & (rd > 0) & (rd > ld)
                    app = tl | tr
                    doa = als[g] & app
                    nwY = jnp.where(tl, spY0, spY1)
                    nwX = jnp.where(tl, spX0, spX1)
                    slot = jnp.clip(ns[g], 0, cap1)
                    gl = g * 16 + lane
                    plsc.store_scatter(ptsC, [slot * stride + gl], nwY, mask=doa)
                    plsc.store_scatter(ptsC, [slot * stride + per_tile + gl], nwX, mask=doa)
                    nn.append(ns[g] + doa.astype(i32))
                    na.append(als[g] & app)
                out[cname] = (tuple(nn), tuple(na))
            return out["L"][0], out["L"][1], out["R"][0], out["R"][1]

        nL, aL, nR, aR = lax.fori_loop(0, _STEPS, step, (tuple(nL0), tuple(aL0), tuple(nR0), tuple(aR0)))

        for g in range(ng):
            tot = nL[g] + nR[g] - 1
            keep = grp(pkv, g) != 0
            lenv[pl.ds(g * 16, 16)] = jnp.where(keep, tot, 0)

        def emit(j, carry):
            jf = jnp.full((16,), j, i32)
            for g in range(ng):
                tot = nL[g] + nR[g] - 1
                keep = grp(pkv, g) != 0
                froml = jf < nL[g]
                slL = jnp.clip(nL[g] - 1 - jf, 0, cap1)
                slR = jnp.clip(jf - nL[g] + 1, 0, cap1)
                gl = g * 16 + lane
                vLy = plsc.load_gather(ptsL, [slL * stride + gl])
                vLx = plsc.load_gather(ptsL, [slL * stride + per_tile + gl])
                vRy = plsc.load_gather(ptsR, [slR * stride + gl])
                vRx = plsc.load_gather(ptsR, [slR * stride + per_tile + gl])
                okm = (jf < tot) & keep
                py_ = jnp.where(okm, jnp.where(froml, vLy, vRy), 0.0)
                px_ = jnp.where(okm, jnp.where(froml, vLx, vRx), 0.0)
                bi = gl * (2 * _LMAX) + j * 2
                plsc.store_scatter(lineb, [bi], py_)
                plsc.store_scatter(lineb, [bi + 1], px_)
            return carry

        lax.fori_loop(0, _LMAX, emit, 0)

        pltpu.sync_copy(lineb, linesr.at[pl.ds(p0 * 2 * _LMAX, per_tile * 2 * _LMAX)])
        pltpu.sync_copy(lenv, lensr.at[pl.ds(p0, per_tile)])

    return tracer


def kernel(segm_logit, side_logit0, side_logit1, center_point, side_points0, side_points1):
    B, H, W, C = segm_logit.shape
    xt = jnp.transpose(segm_logit, (0, 3, 1, 2))
    best_score, best_class, argcls = _dense(xt)

    HW = H * W
    S = 16
    sub_s, sub_i = jax.lax.top_k(best_score.reshape(B * S, HW // S), _K)
    base = (jnp.arange(S, dtype=jnp.int32) * (HW // S)).repeat(_K)
    cand_i = (sub_i.reshape(B, S * _K) + base[None, :]).astype(jnp.int32)
    top_s, tj = jax.lax.top_k(sub_s.reshape(B, S * _K), _K)
    top_i = jnp.take_along_axis(cand_i, tj, axis=1)
    ys = top_i // W
    xs = top_i % W
    cls = jnp.take_along_axis(best_class.reshape(B, H * W), top_i, axis=1)
    valid = top_s > 0.0

    tracer = _sc_trace(H, W, B * _K, B)
    lines_flat, lens_flat = tracer(
        ys.reshape(-1).astype(jnp.int32),
        xs.reshape(-1).astype(jnp.int32),
        cls.reshape(-1).astype(jnp.int32),
        valid.reshape(-1).astype(jnp.int32),
        argcls.reshape(-1),
        center_point[..., 0].reshape(-1),
        center_point[..., 1].reshape(-1),
        side_points0[..., 0].reshape(-1),
        side_points0[..., 1].reshape(-1),
        side_points1[..., 0].reshape(-1),
        side_points1[..., 1].reshape(-1),
        side_logit0.reshape(-1),
        side_logit1.reshape(-1),
    )
    lines = lines_flat.reshape(B, _K, _LMAX, 2)
    lens = lens_flat.reshape(B, _K)
    points = jnp.stack([ys, xs], axis=-1).astype(jnp.int32)
    return (points, cls.astype(jnp.int32), top_s, lines, lens)


# speedup vs baseline: 11.1468x; 1.0400x over previous
import functools

import jax
import jax.numpy as jnp
from jax import lax
from jax.experimental import pallas as pl
from jax.experimental.pallas import tpu as pltpu
from jax.experimental.pallas import tpu_sc as plsc

_K = 512
_STEPS = 16
_LMAX = 40
_CAP = _STEPS + 2


def _peaks_body(x_ref, score_ref, bcls_ref, argc_ref):
    x = x_ref[0]
    C, H, W = x.shape
    m = x[0]
    for c in range(1, C):
        m = jnp.maximum(m, x[c])
    es = []
    s = None
    for c in range(C):
        e = jnp.exp(x[c] - m)
        es.append(e)
        s = e if s is None else s + e
    inv = 1.0 / s
    ninf = jnp.float32(-jnp.inf)
    best = bcls = abest = agc = None
    for c in range(C):
        p = es[c] * inv
        lf = jnp.concatenate([p[:, 1:], jnp.full((H, 1), ninf, p.dtype)], axis=1)
        rt = jnp.concatenate([jnp.full((H, 1), ninf, p.dtype), p[:, :-1]], axis=1)
        mw = jnp.maximum(p, jnp.maximum(lf, rt))
        up = jnp.concatenate([mw[1:], jnp.full((1, W), ninf, p.dtype)], axis=0)
        dn = jnp.concatenate([jnp.full((1, W), ninf, p.dtype), mw[:-1]], axis=0)
        lm = jnp.maximum(mw, jnp.maximum(up, dn))
        pk = (p == lm) & (p > 0.5)
        msk = jnp.where(pk, p, 0.0)
        xc = x[c]
        if best is None:
            best = msk
            bcls = jnp.zeros(msk.shape, jnp.int32)
            abest = xc
            agc = jnp.zeros(msk.shape, jnp.int32)
        else:
            u = msk > best
            bcls = jnp.where(u, c, bcls)
            best = jnp.where(u, msk, best)
            au = xc > abest
            agc = jnp.where(au, c, agc)
            abest = jnp.where(au, xc, abest)
    score_ref[0] = best
    bcls_ref[0] = bcls
    argc_ref[0] = agc


def _dense(xt):
    B, C, H, W = xt.shape
    return pl.pallas_call(
        _peaks_body,
        grid=(B,),
        in_specs=[pl.BlockSpec((1, C, H, W), lambda b: (b, 0, 0, 0))],
        out_specs=[
            pl.BlockSpec((1, H, W), lambda b: (b, 0, 0)),
            pl.BlockSpec((1, H, W), lambda b: (b, 0, 0)),
            pl.BlockSpec((1, H, W), lambda b: (b, 0, 0)),
        ],
        out_shape=[
            jax.ShapeDtypeStruct((B, H, W), jnp.float32),
            jax.ShapeDtypeStruct((B, H, W), jnp.int32),
            jax.ShapeDtypeStruct((B, H, W), jnp.int32),
        ],
    )(xt)


def _side_probe(gp, cid, argcls_b, sp_map, sl_map, H, W):
    gi = jnp.floor(gp).astype(jnp.int32)
    in_g = (gi[0] >= 0) & (gi[0] < H) & (gi[1] >= 0) & (gi[1] < W)
    g0 = jnp.clip(gi[0], 0, H - 1)
    g1 = jnp.clip(gi[1], 0, W - 1)
    sp = gp + 0.5 + sp_map[g0, g1]
    si = jnp.floor(sp).astype(jnp.int32)
    in_s = (si[0] >= 0) & (si[0] < H) & (si[1] >= 0) & (si[1] < W)
    s0 = jnp.clip(si[0], 0, H - 1)
    s1 = jnp.clip(si[1], 0, W - 1)
    ok = in_g & in_s & (argcls_b[s0, s1] == cid) & (sl_map[s0, s1] <= 0.0)
    return sp, ok


def _grow(pts, n, alive, cid, argcls_b, sp_maps, sl_maps, H, W):
    last = pts[jnp.clip(n - 1, 0, _CAP - 1)]
    prev = pts[jnp.clip(n - 2, 0, _CAP - 1)]
    direction = last - prev
    lp, lok = _side_probe(last, cid, argcls_b, sp_maps[0], sl_maps[0], H, W)
    rp, rok = _side_probe(last, cid, argcls_b, sp_maps[1], sl_maps[1], H, W)
    ld = jnp.where(lok, lp[0] * direction[0] + lp[1] * direction[1], -1.0)
    rd = jnp.where(rok, rp[0] * direction[0] + rp[1] * direction[1], -1.0)
    take_l = lok & (ld > 0) & (ld > rd)
    take_r = rok & (rd > 0) & (rd > ld)
    appended = take_l | take_r
    do_append = alive & appended
    new_pt = jnp.where(take_l, lp, rp)
    slot = jnp.clip(n, 0, _CAP - 1)
    pts = pts.at[slot].set(jnp.where(do_append, new_pt, pts[slot]))
    n = n + jnp.where(do_append, 1, 0).astype(jnp.int32)
    alive = alive & appended
    return pts, n, alive


def _trace(ys, xs, cls, valid, argcls, cp, sp0, sp1, sl0, sl1, H, W):
    def trace_peak(py, px, cid, keep, argcls_b, cp_b, sp0_b, sp1_b, sl0_b, sl1_b):
        gp = jnp.stack([py, px]).astype(jnp.float32)
        start = gp + cp_b[py, px]
        lpt, lok = _side_probe(gp, cid, argcls_b, sp0_b, sl0_b, H, W)
        rpt, rok = _side_probe(gp, cid, argcls_b, sp1_b, sl1_b, H, W)

        def mk(p0, ok0):
            pts = jnp.zeros((_CAP, 2), jnp.float32)
            pts = pts.at[0].set(start)
            pts = pts.at[1].set(jnp.where(ok0, p0, 0.0))
            n = jnp.where(ok0, 2, 1).astype(jnp.int32)
            return pts, n, ok0

        def body(_, carry):
            (a_pts, a_n, a_al), (b_pts, b_n, b_al) = carry
            left = _grow(a_pts, a_n, a_al, cid, argcls_b, (sp0_b, sp1_b), (sl0_b, sl1_b), H, W)
            right = _grow(b_pts, b_n, b_al, cid, argcls_b, (sp0_b, sp1_b), (sl0_b, sl1_b), H, W)
            return left, right

        (pl_, nl, _), (pr_, nr, _) = jax.lax.fori_loop(
            0, _STEPS, body, (mk(lpt, lok), mk(rpt, rok))
        )
        total = nl + nr - 1
        j = jnp.arange(_LMAX)
        from_l = j < nl
        pt = jnp.where(
            from_l[:, None],
            pl_[jnp.clip(nl - 1 - j, 0, _CAP - 1)],
            pr_[jnp.clip(j - nl + 1, 0, _CAP - 1)],
        )
        line = jnp.where((j < total)[:, None] & keep, pt, 0.0)
        length = jnp.where(keep, total, 0).astype(jnp.int32)
        return line, length

    per_peak = jax.vmap(trace_peak, in_axes=(0, 0, 0, 0, None, None, None, None, None, None))
    per_batch = jax.vmap(per_peak)
    return per_batch(ys, xs, cls, valid, argcls, cp, sp0, sp1, sl0, sl1)


def _sc_trace(H, W, npeak, nbatch):
    HW = H * W
    per_tile = npeak // 32
    ng = per_tile // 16
    cap1 = _CAP - 1

    mesh = plsc.VectorSubcoreMesh(core_axis_name="c", subcore_axis_name="s")
    f32 = jnp.float32
    i32 = jnp.int32

    @functools.partial(
        pl.kernel,
        mesh=mesh,
        compiler_params=pltpu.CompilerParams(needs_layout_passes=False),
        out_type=[
            jax.ShapeDtypeStruct((npeak * 2 * _LMAX,), f32),
            jax.ShapeDtypeStruct((npeak,), i32),
        ],
        scratch_types=(
            [pltpu.VMEM((per_tile,), i32) for _ in range(4)]
            + [pltpu.VMEM((per_tile,), i32) for _ in range(6)]
            + [pltpu.VMEM((per_tile,), f32) for _ in range(8)]
            + [pltpu.VMEM((per_tile,), i32) for _ in range(4)]
            + [pltpu.VMEM((per_tile,), f32) for _ in range(4)]
            + [pltpu.VMEM((_CAP * 2 * per_tile,), f32) for _ in range(2)]
            + [pltpu.VMEM((per_tile * 2 * _LMAX,), f32)]
            + [pltpu.VMEM((per_tile,), i32)]
            + [pltpu.SemaphoreType.DMA]
        ),
    )
    def tracer(ysr, xsr, clsr, vldr, acr, cpyr, cpxr, s0yr, s0xr, s1yr, s1xr,
               sl0r, sl1r, linesr, lensr,
               pky, pkx, pkc, pkv, ixL, ixR, jA, jB, jC, jD,
               ay0, ax0, ay1, ax1, by0, bx0, by1, bx1,
               cA, cB, cC, cD, sA, sB, sC2, sD,
               ptsL, ptsR, lineb, lenv, sem):
        lane = lax.broadcasted_iota(i32, (16,), 0)

        def fl(x):
            t = x.astype(i32)
            return jnp.where(x < t.astype(f32), t - 1, t)

        def grp(ref, g):
            return ref[pl.ds(g * 16, 16)]

        def drain(descs):
            for d in descs:
                d.wait()

        wid = lax.axis_index("c") * 16 + lax.axis_index("s")
        p0 = wid * per_tile
        moff = (wid // (32 // nbatch)) * HW

        pltpu.sync_copy(ysr.at[pl.ds(p0, per_tile)], pky)
        pltpu.sync_copy(xsr.at[pl.ds(p0, per_tile)], pkx)
        pltpu.sync_copy(clsr.at[pl.ds(p0, per_tile)], pkc)
        pltpu.sync_copy(vldr.at[pl.ds(p0, per_tile)], pkv)

        for g in range(ng):
            py = grp(pky, g)
            px = grp(pkx, g)
            ixL[pl.ds(g * 16, 16)] = moff + py * W + px
        drain([
            pltpu.async_copy(cpyr.at[ixL], ay0, sem),
            pltpu.async_copy(cpxr.at[ixL], ax0, sem),
            pltpu.async_copy(s0yr.at[ixL], ay1, sem),
            pltpu.async_copy(s0xr.at[ixL], ax1, sem),
            pltpu.async_copy(s1yr.at[ixL], by0, sem),
            pltpu.async_copy(s1xr.at[ixL], bx0, sem),
        ])
        init = []
        for g in range(ng):
            fy = grp(pky, g).astype(f32)
            fx = grp(pkx, g).astype(f32)
            sty = fy + grp(ay0, g)
            stx = fx + grp(ax0, g)
            lpY = fy + 0.5 + grp(ay1, g)
            lpX = fx + 0.5 + grp(ax1, g)
            rpY = fy + 0.5 + grp(by0, g)
            rpX = fx + 0.5 + grp(bx0, g)
            liy = fl(lpY)
            lix = fl(lpX)
            lin = (liy >= 0) & (liy < H) & (lix >= 0) & (lix < W)
            jA[pl.ds(g * 16, 16)] = moff + jnp.clip(liy, 0, H - 1) * W + jnp.clip(lix, 0, W - 1)
            riy = fl(rpY)
            rix = fl(rpX)
            rin = (riy >= 0) & (riy < H) & (rix >= 0) & (rix < W)
            jB[pl.ds(g * 16, 16)] = moff + jnp.clip(riy, 0, H - 1) * W + jnp.clip(rix, 0, W - 1)
            init.append((sty, stx, lpY, lpX, rpY, rpX, lin, rin))
        drain([
            pltpu.async_copy(acr.at[jA], cA, sem),
            pltpu.async_copy(sl0r.at[jA], sA, sem),
            pltpu.async_copy(acr.at[jB], cB, sem),
            pltpu.async_copy(sl1r.at[jB], sB, sem),
        ])
        nL0, aL0, nR0, aR0 = [], [], [], []
        for g in range(ng):
            sty, stx, lpY, lpX, rpY, rpX, lin, rin = init[g]
            cid = grp(pkc, g)
            lok = lin & (grp(cA, g) == cid) & (grp(sA, g) <= 0.0)
            rok = rin & (grp(cB, g) == cid) & (grp(sB, g) <= 0.0)
            ptsL[pl.ds(g * 16, 16)] = sty
            ptsL[pl.ds(per_tile + g * 16, 16)] = stx
            ptsL[pl.ds(2 * per_tile + g * 16, 16)] = jnp.where(lok, lpY, 0.0)
            ptsL[pl.ds(3 * per_tile + g * 16, 16)] = jnp.where(lok, lpX, 0.0)
            ptsR[pl.ds(g * 16, 16)] = sty
            ptsR[pl.ds(per_tile + g * 16, 16)] = stx
            ptsR[pl.ds(2 * per_tile + g * 16, 16)] = jnp.where(rok, rpY, 0.0)
            ptsR[pl.ds(3 * per_tile + g * 16, 16)] = jnp.where(rok, rpX, 0.0)
            nL0.append(jnp.where(lok, 2, 1).astype(i32))
            aL0.append(lok)
            nR0.append(jnp.where(rok, 2, 1).astype(i32))
            aR0.append(rok)

        stride = 2 * per_tile

        def step(_, carry):
            nL, aL, nR, aR = carry
            stash = {}
            for cname, ns, ix, ptsC in (("L", nL, ixL, ptsL), ("R", nR, ixR, ptsR)):
                for g in range(ng):
                    n = ns[g]
                    s1_ = jnp.clip(n - 1, 0, cap1)
                    s2_ = jnp.clip(n - 2, 0, cap1)
                    gl = g * 16 + lane
                    lastY = plsc.load_gather(ptsC, [s1_ * stride + gl])
                    lastX = plsc.load_gather(ptsC, [s1_ * stride + per_tile + gl])
                    prevY = plsc.load_gather(ptsC, [s2_ * stride + gl])
                    prevX = plsc.load_gather(ptsC, [s2_ * stride + per_tile + gl])
                    dY = lastY - prevY
                    dX = lastX - prevX
                    giy = fl(lastY)
                    gix = fl(lastX)
                    ing = (giy >= 0) & (giy < H) & (gix >= 0) & (gix < W)
                    ix[pl.ds(g * 16, 16)] = (
                        moff + jnp.clip(giy, 0, H - 1) * W + jnp.clip(gix, 0, W - 1)
                    )
                    stash[(cname, g)] = (lastY, lastX, dY, dX, ing)
            drain([
                pltpu.async_copy(s0yr.at[ixL], ay0, sem),
                pltpu.async_copy(s0xr.at[ixL], ax0, sem),
                pltpu.async_copy(s1yr.at[ixL], ay1, sem),
                pltpu.async_copy(s1xr.at[ixL], ax1, sem),
                pltpu.async_copy(s0yr.at[ixR], by0, sem),
                pltpu.async_copy(s0xr.at[ixR], bx0, sem),
                pltpu.async_copy(s1yr.at[ixR], by1, sem),
                pltpu.async_copy(s1xr.at[ixR], bx1, sem),
            ])
            stash2 = {}
            for cname, srcs, jref in (
                ("L0", (ay0, ax0), jA),
                ("L1", (ay1, ax1), jB),
                ("R0", (by0, bx0), jC),
                ("R1", (by1, bx1), jD),
            ):
                for g in range(ng):
                    lastY, lastX, dY, dX, ing = stash[(cname[0], g)]
                    spY = lastY + 0.5 + grp(srcs[0], g)
                    spX = lastX + 0.5 + grp(srcs[1], g)
                    siy = fl(spY)
                    six = fl(spX)
                    ins = ing & (siy >= 0) & (siy < H) & (six >= 0) & (six < W)
                    jref[pl.ds(g * 16, 16)] = (
                        moff + jnp.clip(siy, 0, H - 1) * W + jnp.clip(six, 0, W - 1)
                    )
                    stash2[(cname, g)] = (spY, spX, ins)
            drain([
                pltpu.async_copy(acr.at[jA], cA, sem),
                pltpu.async_copy(sl0r.at[jA], sA, sem),
                pltpu.async_copy(acr.at[jB], cB, sem),
                pltpu.async_copy(sl1r.at[jB], sB, sem),
                pltpu.async_copy(acr.at[jC], cC, sem),
                pltpu.async_copy(sl0r.at[jC], sC2, sem),
                pltpu.async_copy(acr.at[jD], cD, sem),
                pltpu.async_copy(sl1r.at[jD], sD, sem),
            ])
            out = {}
            for cname, ns, als, ptsC, refs0, refs1 in (
                ("L", nL, aL, ptsL, (cA, sA), (cB, sB)),
                ("R", nR, aR, ptsR, (cC, sC2), (cD, sD)),
            ):
                nn, na = [], []
                for g in range(ng):
                    cid = grp(pkc, g)
                    lastY, lastX, dY, dX, ing = stash[(cname, g)]
                    spY0, spX0, ins0 = stash2[(cname + "0", g)]
                    spY1, spX1, ins1 = stash2[(cname + "1", g)]
                    lok = ins0 & (grp(refs0[0], g) == cid) & (grp(refs0[1], g) <= 0.0)
                    rok = ins1 & (grp(refs1[0], g) == cid) & (grp(refs1[1], g) <= 0.0)
                    ld = jnp.where(lok, spY0 * dY + spX0 * dX, -1.0)
                    rd = jnp.where(rok, spY1 * dY + spX1 * dX, -1.0)
                    tl = lok & (ld > 0) & (ld > rd)
                    tr = rok & (rd > 0) & (rd > ld)
                    app = tl | tr
                    doa = als[g] & app
                    nwY = jnp.where(tl, spY0, spY1)
                    nwX = jnp.where(tl, spX0, spX1)
                    slot = jnp.clip(ns[g], 0, cap1)
                    gl = g * 16 + lane
                    plsc.store_scatter(ptsC, [slot * stride + gl], nwY, mask=doa)
                    plsc.store_scatter(ptsC, [slot * stride + per_tile + gl], nwX, mask=doa)
                    nn.append(ns[g] + doa.astype(i32))
                    na.append(als[g] & app)
                out[cname] = (tuple(nn), tuple(na))
            return out["L"][0], out["L"][1], out["R"][0], out["R"][1]

        nL, aL, nR, aR = lax.fori_loop(0, _STEPS, step, (tuple(nL0), tuple(aL0), tuple(nR0), tuple(aR0)))

        for g in range(ng):
            tot = nL[g] + nR[g] - 1
            keep = grp(pkv, g) != 0
            lenv[pl.ds(g * 16, 16)] = jnp.where(keep, tot, 0)

        def emit(j, carry):
            jf = jnp.full((16,), j, i32)
            for g in range(ng):
                tot = nL[g] + nR[g] - 1
                keep = grp(pkv, g) != 0
                froml = jf < nL[g]
                slL = jnp.clip(nL[g] - 1 - jf, 0, cap1)
                slR = jnp.clip(jf - nL[g] + 1, 0, cap1)
                gl = g * 16 + lane
                vLy = plsc.load_gather(ptsL, [slL * stride + gl])
                vLx = plsc.load_gather(ptsL, [slL * stride + per_tile + gl])
                vRy = plsc.load_gather(ptsR, [slR * stride + gl])
                vRx = plsc.load_gather(ptsR, [slR * stride + per_tile + gl])
                okm = (jf < tot) & keep
                py_ = jnp.where(okm, jnp.where(froml, vLy, vRy), 0.0)
                px_ = jnp.where(okm, jnp.where(froml, vLx, vRx), 0.0)
                bi = gl * (2 * _LMAX) + j * 2
                plsc.store_scatter(lineb, [bi], py_)
                plsc.store_scatter(lineb, [bi + 1], px_)
            return carry

        lax.fori_loop(0, _LMAX, emit, 0)

        pltpu.sync_copy(lineb, linesr.at[pl.ds(p0 * 2 * _LMAX, per_tile * 2 * _LMAX)])
        pltpu.sync_copy(lenv, lensr.at[pl.ds(p0, per_tile)])

    return tracer


def kernel(segm_logit, side_logit0, side_logit1, center_point, side_points0, side_points1):
    B, H, W, C = segm_logit.shape
    xt = jnp.transpose(segm_logit, (0, 3, 1, 2))
    best_score, best_class, argcls = _dense(xt)

    top_s, top_i = jax.lax.top_k(best_score.reshape(B, H * W), _K)
    ys = top_i // W
    xs = top_i % W
    cls = jnp.take_along_axis(best_class.reshape(B, H * W), top_i, axis=1)
    valid = top_s > 0.0

    tracer = _sc_trace(H, W, B * _K, B)
    lines_flat, lens_flat = tracer(
        ys.reshape(-1).astype(jnp.int32),
        xs.reshape(-1).astype(jnp.int32),
        cls.reshape(-1).astype(jnp.int32),
        valid.reshape(-1).astype(jnp.int32),
        argcls.reshape(-1),
        center_point[..., 0].reshape(-1),
        center_point[..., 1].reshape(-1),
        side_points0[..., 0].reshape(-1),
        side_points0[..., 1].reshape(-1),
        side_points1[..., 0].reshape(-1),
        side_points1[..., 1].reshape(-1),
        side_logit0.reshape(-1),
        side_logit1.reshape(-1),
    )
    lines = lines_flat.reshape(B, _K, _LMAX, 2)
    lens = lens_flat.reshape(B, _K)
    points = jnp.stack([ys, xs], axis=-1).astype(jnp.int32)
    return (points, cls.astype(jnp.int32), top_s, lines, lens)
